# baseline (speedup 1.0000x reference)
import functools

import jax
import jax.numpy as jnp
from jax import lax
from jax.experimental import pallas as pl
from jax.experimental.pallas import tpu as pltpu
from jax.experimental.pallas import tpu_sc as plsc

VOCAB = 100000
D = 64
BATCH = 4096
TEXT_LEN = 200
TOPIC_LEN = 20

NC = 2
NS = 16
NW = NC * NS

CHUNK = 128
K1 = 5
K2 = 2

B1 = BATCH * TEXT_LEN
B2 = BATCH * TOPIC_LEN
PW1 = B1 // NW
PW2 = B2 // NW
NCH1 = PW1 // CHUNK
NCH2 = PW2 // CHUNK


def _make_kernel(nrows, nch, kk):
    mesh = plsc.VectorSubcoreMesh(core_axis_name="c", subcore_axis_name="s")
    pw = nrows // NW

    @functools.partial(
        pl.kernel,
        mesh=mesh,
        compiler_params=pltpu.CompilerParams(use_tc_tiling_on_sc=False),
        out_type=jax.ShapeDtypeStruct((nrows, D), jnp.float32),
        scratch_types=[
            pltpu.VMEM((nch, CHUNK), jnp.int32),
            pltpu.VMEM((2, kk * CHUNK, D), jnp.float32),
            [pltpu.SemaphoreType.DMA] * 2,
            [pltpu.SemaphoreType.DMA] * 2,
        ],
    )
    def k(table, indices, out_arr, idx_v, rows, gsem, osem):
        wid = lax.axis_index("s") * NC + lax.axis_index("c")

        pltpu.sync_copy(indices.at[wid], idx_v)

        def run(idx, nch, out, base, kk):
            ngroups = nch // kk

            def fire_g(t, s):
                for i in range(kk):
                    pltpu.async_copy(
                        table.at[idx.at[t * kk + i]],
                        rows.at[s, pl.ds(i * CHUNK, CHUNK)],
                        gsem[s],
                    )

            def drain_g(t, s):
                for i in range(kk):
                    pltpu.make_async_copy(
                        table.at[idx.at[t * kk + i]],
                        rows.at[s, pl.ds(i * CHUNK, CHUNK)],
                        gsem[s],
                    ).wait()

            def wb(t, s):
                return pltpu.make_async_copy(
                    rows.at[s, pl.ds(0, kk * CHUNK)],
                    out.at[pl.ds(base + t * kk * CHUNK, kk * CHUNK)],
                    osem[s],
                )

            fire_g(0, 0)
            fire_g(1, 1)
            drain_g(0, 0)
            wb(0, 0).start()

            def body(p, _):
                t0 = 2 + 2 * p
                drain_g(t0 - 1, 1)
                wb(t0 - 1, 1).start()
                wb(t0 - 2, 0).wait()
                fire_g(t0, 0)
                drain_g(t0, 0)
                wb(t0, 0).start()
                wb(t0 - 1, 1).wait()
                fire_g(t0 + 1, 1)
                return 0

            lax.fori_loop(0, (ngroups - 2) // 2, body, 0)

            drain_g(ngroups - 1, 1)
            wb(ngroups - 1, 1).start()
            wb(ngroups - 2, 0).wait()
            wb(ngroups - 1, 1).wait()

        run(idx_v, nch, out_arr, wid * pw, kk)

    return k


_kern_text = _make_kernel(B1, NCH1, K1)
_kern_topic = _make_kernel(B2, NCH2, K2)


def kernel(table, text, topic):
    text_r = text.reshape(NW, NCH1, CHUNK).astype(jnp.int32)
    topic_r = topic.reshape(NW, NCH2, CHUNK).astype(jnp.int32)
    out2 = _kern_topic(table, topic_r)
    out1 = _kern_text(table, text_r)
    return (
        out1.reshape(BATCH, TEXT_LEN, D),
        out2.reshape(BATCH, TOPIC_LEN, D),
    )


# speedup vs baseline: 1.0010x; 1.0010x over previous
import functools

import jax
import jax.numpy as jnp
from jax import lax
from jax.experimental import pallas as pl
from jax.experimental.pallas import tpu as pltpu
from jax.experimental.pallas import tpu_sc as plsc

VOCAB = 100000
D = 64
BATCH = 4096
TEXT_LEN = 200
TOPIC_LEN = 20

NC = 2
NS = 16
NW = NC * NS

CHUNK = 128
K1 = 4
K2 = 2

B1 = BATCH * TEXT_LEN
B2 = BATCH * TOPIC_LEN
PW1 = B1 // NW
PW2 = B2 // NW
NCH1 = PW1 // CHUNK
NCH2 = PW2 // CHUNK


def _make_kernel(nrows, nch, kk):
    mesh = plsc.VectorSubcoreMesh(core_axis_name="c", subcore_axis_name="s")
    pw = nrows // NW

    @functools.partial(
        pl.kernel,
        mesh=mesh,
        compiler_params=pltpu.CompilerParams(use_tc_tiling_on_sc=False),
        out_type=jax.ShapeDtypeStruct((nrows, D), jnp.float32),
        scratch_types=[
            pltpu.VMEM((nch, CHUNK), jnp.int32),
            pltpu.VMEM((2, kk * CHUNK, D), jnp.float32),
            [pltpu.SemaphoreType.DMA] * 2,
            [pltpu.SemaphoreType.DMA] * 2,
        ],
    )
    def k(table, indices, out_arr, idx_v, rows, gsem, osem):
        wid = lax.axis_index("s") * NC + lax.axis_index("c")

        pltpu.sync_copy(indices.at[wid], idx_v)

        def run(idx, nch, out, base, kk):
            ngroups = nch // kk

            def fire_g(t, s):
                for i in range(kk):
                    pltpu.async_copy(
                        table.at[idx.at[t * kk + i]],
                        rows.at[s, pl.ds(i * CHUNK, CHUNK)],
                        gsem[s],
                    )

            def drain_g(t, s):
                for i in range(kk):
                    pltpu.make_async_copy(
                        table.at[idx.at[t * kk + i]],
                        rows.at[s, pl.ds(i * CHUNK, CHUNK)],
                        gsem[s],
                    ).wait()

            def wb(t, s):
                return pltpu.make_async_copy(
                    rows.at[s, pl.ds(0, kk * CHUNK)],
                    out.at[pl.ds(base + t * kk * CHUNK, kk * CHUNK)],
                    osem[s],
                )

            fire_g(0, 0)
            fire_g(1, 1)
            drain_g(0, 0)
            wb(0, 0).start()

            def body(p, _):
                t0 = 2 + 2 * p
                drain_g(t0 - 1, 1)
                wb(t0 - 1, 1).start()
                wb(t0 - 2, 0).wait()
                fire_g(t0, 0)
                drain_g(t0, 0)
                wb(t0, 0).start()
                wb(t0 - 1, 1).wait()
                fire_g(t0 + 1, 1)
                return 0

            lax.fori_loop(0, (ngroups - 2) // 2, body, 0)

            drain_g(ngroups - 1, 1)
            wb(ngroups - 1, 1).start()
            wb(ngroups - 2, 0).wait()
            wb(ngroups - 1, 1).wait()

        run(idx_v, nch, out_arr, wid * pw, kk)

    return k


_kern_text = _make_kernel(B1, NCH1, K1)
_kern_topic = _make_kernel(B2, NCH2, K2)


def kernel(table, text, topic):
    text_r = text.reshape(NW, NCH1, CHUNK).astype(jnp.int32)
    topic_r = topic.reshape(NW, NCH2, CHUNK).astype(jnp.int32)
    out2 = _kern_topic(table, topic_r)
    out1 = _kern_text(table, text_r)
    return (
        out1.reshape(BATCH, TEXT_LEN, D),
        out2.reshape(BATCH, TOPIC_LEN, D),
    )
